# baseline (speedup 1.0000x reference)
import functools

import jax
import jax.numpy as jnp
from jax import lax
from jax.experimental import pallas as pl
from jax.experimental.pallas import tpu as pltpu
from jax.experimental.pallas import tpu_sc as plsc

_INFO = plsc.get_sparse_core_info()
_NC, _NS = _INFO.num_cores, _INFO.num_subcores
_NW = _NC * _NS
_L = 16
_PITCH = 129


@functools.lru_cache(maxsize=None)
def _build_transpose(vocab, n_embd):
    full_blocks = vocab // 128
    tail = vocab - full_blocks * 128
    k_per_w = full_blocks // _NW
    k_rem = full_blocks - k_per_w * _NW
    nbuf = 4
    assert k_per_w % nbuf == 0

    mesh = plsc.VectorSubcoreMesh(core_axis_name="c", subcore_axis_name="s")

    @functools.partial(
        pl.kernel,
        out_type=jax.ShapeDtypeStruct((vocab * n_embd // 128, 128), jnp.float32),
        mesh=mesh,
        scratch_types=[
            pltpu.VMEM((nbuf, n_embd, _PITCH), jnp.float32),
            pltpu.VMEM((nbuf, n_embd, 128), jnp.float32),
        ]
        + [pltpu.SemaphoreType.DMA] * (2 * nbuf),
        compiler_params=pltpu.CompilerParams(
            use_tc_tiling_on_sc=True, needs_layout_passes=False
        ),
    )
    def transpose(tbl_t, tail2, out_hbm, bin_, bout, *sems):
        sin, sout = sems[:nbuf], sems[nbuf:]
        wid = lax.axis_index("s") * _NC + lax.axis_index("c")
        iota = lax.iota(jnp.int32, _L)
        zero = jnp.full((_L,), 0, jnp.int32)

        feats = [iota + dc * _L for dc in range(n_embd // _L)]

        def start_in(j, b):
            pltpu.async_copy(
                tbl_t.at[:, pl.ds(j * 128, 128)],
                bin_.at[b].at[:, pl.ds(0, 128)],
                sin[b],
            )

        def wait_in(b):
            pltpu.make_async_copy(
                tbl_t.at[:, pl.ds(0, 128)],
                bin_.at[b].at[:, pl.ds(0, 128)],
                sin[b],
            ).wait()

        def start_out(j, b):
            pltpu.async_copy(
                bout.at[b], out_hbm.at[pl.ds(j * 64, 64), :], sout[b]
            )

        def wait_out(b):
            pltpu.make_async_copy(
                out_hbm.at[pl.ds(0, 64), :], bout.at[b], sout[b]
            ).wait()

        def transpose_block(b):
            @plsc.parallel_loop(0, 64, unroll=16)
            def _(r):
                jv0 = zero + 2 * r
                jv1 = jv0 + 1
                for dc in range(n_embd // _L):
                    v0 = plsc.load_gather(bin_.at[b], [feats[dc], jv0])
                    v1 = plsc.load_gather(bin_.at[b], [feats[dc], jv1])
                    bout.at[b][r, pl.ds(dc * _L, _L)] = v0
                    bout.at[b][r, pl.ds(n_embd + dc * _L, _L)] = v1

        def blk(k):
            return wid + k * _NW

        for b in range(nbuf):
            start_in(blk(b), b)

        @pl.loop(0, k_per_w, step=nbuf)
        def _(k0):
            for b in range(nbuf):
                k = k0 + b
                wait_in(b)

                @pl.when(k >= nbuf)
                def _():
                    wait_out(b)

                transpose_block(b)
                start_out(blk(k), b)
                nk = k + nbuf

                @pl.when(nk < k_per_w)
                def _():
                    start_in(blk(nk), b)

        for b in range(nbuf):
            wait_out(b)

        @pl.when(wid < k_rem)
        def _():
            j = k_per_w * _NW + wid
            pltpu.async_copy(
                tbl_t.at[:, pl.ds(j * 128, 128)],
                bin_.at[0].at[:, pl.ds(0, 128)],
                sin[0],
            ).wait()
            transpose_block(0)
            pltpu.async_copy(
                bout.at[0], out_hbm.at[pl.ds(j * 64, 64), :], sout[0]
            ).wait()

        if tail:
            @pl.when(wid == k_rem)
            def _():
                pltpu.async_copy(
                    tail2, bout.at[0].at[pl.ds(0, tail // 2), :], sin[0]
                ).wait()
                pltpu.async_copy(
                    bout.at[0].at[pl.ds(0, tail // 2), :],
                    out_hbm.at[pl.ds(full_blocks * 64, tail // 2), :],
                    sout[0],
                ).wait()

    return transpose


@functools.lru_cache(maxsize=None)
def _build_gather(vocab, n_embd, n_tokens):
    assert n_tokens % _NW == 0
    per_w = n_tokens // _NW
    chunk = 128
    assert per_w % chunk == 0
    n_chunks = per_w // chunk
    nbuf = 8
    assert n_chunks % nbuf == 0

    mesh = plsc.VectorSubcoreMesh(core_axis_name="c", subcore_axis_name="s")

    @functools.partial(
        pl.kernel,
        out_type=jax.ShapeDtypeStruct((n_tokens, 128), jnp.float32),
        mesh=mesh,
        scratch_types=[
            pltpu.VMEM((n_chunks, chunk), jnp.int32),
            pltpu.VMEM((nbuf, chunk, n_embd), jnp.float32),
        ]
        + [pltpu.SemaphoreType.DMA] * nbuf,
        compiler_params=pltpu.CompilerParams(use_tc_tiling_on_sc=False),
    )
    def emb(table_hbm, idx_hbm, out_hbm, idx_v, bufs, *sems):
        wid = lax.axis_index("s") * _NC + lax.axis_index("c")
        base = wid * per_w
        pltpu.sync_copy(idx_hbm.at[wid], idx_v)

        def start(j, b):
            pltpu.async_copy(table_hbm.at[idx_v.at[j]], bufs.at[b], sems[b])

        def wait(b):
            pltpu.make_async_copy(
                table_hbm.at[pl.ds(0, chunk)], bufs.at[b], sems[b]
            ).wait()

        for b in range(nbuf):
            start(b, b)

        @pl.loop(0, n_chunks, step=nbuf)
        def _(j0):
            for b in range(nbuf):
                j = j0 + b
                wait(b)
                pltpu.sync_copy(
                    bufs.at[b],
                    out_hbm.at[pl.ds(base + j * chunk, chunk), pl.ds(0, n_embd)],
                )
                nxt = j + nbuf

                @pl.when(nxt < n_chunks)
                def _():
                    start(nxt, b)

    return emb


def kernel(token_ids, emb_weight):
    b, t = token_ids.shape
    vocab, n_embd = emb_weight.shape
    n_tokens = b * t
    transpose = _build_transpose(vocab, n_embd)
    emb = _build_gather(vocab, n_embd, n_tokens)
    full = (vocab // 128) * 128
    if full < vocab:
        tail2 = emb_weight[full:].reshape(-1, 128)
    else:
        tail2 = jnp.zeros((8, 128), jnp.float32)
    table_rm = transpose(emb_weight.T, tail2)
    table = table_rm.reshape(vocab, n_embd)
    idx = token_ids.astype(jnp.int32).reshape(_NW, -1, 128)
    out_pad = emb(table, idx)
    return out_pad[:, :n_embd].reshape(b, t, n_embd)


# speedup vs baseline: 1.2583x; 1.2583x over previous
import functools

import jax
import jax.numpy as jnp
from jax import lax
from jax.experimental import pallas as pl
from jax.experimental.pallas import tpu as pltpu
from jax.experimental.pallas import tpu_sc as plsc

_INFO = plsc.get_sparse_core_info()
_NC, _NS = _INFO.num_cores, _INFO.num_subcores
_NW = _NC * _NS


@functools.lru_cache(maxsize=None)
def _build(vocab, n_embd, n_tokens):
    assert n_tokens % _NW == 0
    per_w = n_tokens // _NW
    chunk = 128
    assert per_w % chunk == 0
    n_chunks = per_w // chunk
    nbuf = 8
    assert n_chunks % nbuf == 0

    mesh = plsc.VectorSubcoreMesh(core_axis_name="c", subcore_axis_name="s")

    @functools.partial(
        pl.kernel,
        out_type=jax.ShapeDtypeStruct((n_tokens, 128), jnp.float32),
        mesh=mesh,
        scratch_types=[
            pltpu.VMEM((n_chunks, chunk), jnp.int32),
            pltpu.VMEM((nbuf, chunk, n_embd), jnp.float32),
        ]
        + [pltpu.SemaphoreType.DMA] * nbuf,
        compiler_params=pltpu.CompilerParams(use_tc_tiling_on_sc=False),
    )
    def emb(table_hbm, idx_hbm, out_hbm, idx_v, bufs, *sems):
        wid = lax.axis_index("s") * _NC + lax.axis_index("c")
        base = wid * per_w
        pltpu.sync_copy(idx_hbm.at[wid], idx_v)

        def start(j, b):
            pltpu.async_copy(table_hbm.at[idx_v.at[j]], bufs.at[b], sems[b])

        def wait(b):
            pltpu.make_async_copy(
                table_hbm.at[pl.ds(0, chunk)], bufs.at[b], sems[b]
            ).wait()

        for b in range(nbuf):
            start(b, b)

        @pl.loop(0, n_chunks, step=nbuf)
        def _(j0):
            for b in range(nbuf):
                j = j0 + b
                wait(b)
                pltpu.sync_copy(
                    bufs.at[b],
                    out_hbm.at[pl.ds(base + j * chunk, chunk), pl.ds(0, n_embd)],
                )
                nxt = j + nbuf

                @pl.when(nxt < n_chunks)
                def _():
                    start(nxt, b)

    return emb


def kernel(token_ids, emb_weight):
    b, t = token_ids.shape
    vocab, n_embd = emb_weight.shape
    n_tokens = b * t
    emb = _build(vocab, n_embd, n_tokens)
    idx = token_ids.astype(jnp.int32).reshape(_NW, -1, 128)
    out_pad = emb(emb_weight, idx)
    return out_pad[:, :n_embd].reshape(b, t, n_embd)
